# baseline (speedup 1.0000x reference)
import functools
import math

import jax
import jax.numpy as jnp
from jax import lax
from jax.experimental import pallas as pl
from jax.experimental.pallas import tpu as pltpu
from jax.experimental.pallas import tpu_sc as plsc

D_MODEL = 64
SCALE = math.sqrt(D_MODEL)


def _make_sc_lookup(B: int, D: int):
  info = plsc.get_sparse_core_info()
  NC, NS, L = info.num_cores, info.num_subcores, info.num_lanes
  NW = NC * NS
  assert B % NW == 0 and D % L == 0
  b_per_w = B // NW
  CHUNK = 512
  assert b_per_w % CHUNK == 0
  n_chunks = b_per_w // CHUNK
  d_vecs = D // L

  mesh = plsc.VectorSubcoreMesh(core_axis_name="c", subcore_axis_name="s")

  @functools.partial(
      pl.kernel,
      out_type=jax.ShapeDtypeStruct((B, D), jnp.float32),
      mesh=mesh,
      scratch_types=[
          pltpu.VMEM((CHUNK,), jnp.int32),
          pltpu.VMEM((CHUNK, D), jnp.float32),
          pltpu.SemaphoreType.DMA,
      ],
      compiler_params=pltpu.CompilerParams(use_tc_tiling_on_sc=False),
  )
  def lookup(idx_hbm, table_hbm, out_hbm, idx_v, rows_v, sem):
    wid = lax.axis_index("s") * NC + lax.axis_index("c")
    base = wid * b_per_w

    def chunk_body(g, carry):
      off = base + g * CHUNK
      pltpu.sync_copy(idx_hbm.at[pl.ds(off, CHUNK)], idx_v)
      pltpu.async_copy(table_hbm.at[idx_v], rows_v, sem).wait()

      def scale_body(i, c):
        for j in range(d_vecs):
          sl = pl.ds(j * L, L)
          rows_v[i, sl] = rows_v[i, sl] * SCALE
        return c

      lax.fori_loop(0, CHUNK, scale_body, 0)
      pltpu.sync_copy(rows_v, out_hbm.at[pl.ds(off, CHUNK)])
      return carry

    lax.fori_loop(0, n_chunks, chunk_body, 0)

  return lookup


def kernel(x, table):
  S0, S1 = x.shape
  V, D = table.shape
  B = S0 * S1
  idx = x.reshape(B).astype(jnp.int32)
  out = _make_sc_lookup(B, D)(idx, table)
  return out.reshape(S0, S1, D)


# speedup vs baseline: 1.1321x; 1.1321x over previous
import functools
import math

import jax
import jax.numpy as jnp
from jax import lax
from jax.experimental import pallas as pl
from jax.experimental.pallas import tpu as pltpu
from jax.experimental.pallas import tpu_sc as plsc

D_MODEL = 64
SCALE = math.sqrt(D_MODEL)
NBUF = 2
CHUNK = 640


def _make_sc_lookup(B: int, D: int):
  info = plsc.get_sparse_core_info()
  NC, NS, L = info.num_cores, info.num_subcores, info.num_lanes
  NW = NC * NS
  assert B % NW == 0 and D % L == 0
  b_per_w = B // NW
  assert b_per_w % (CHUNK * NBUF) == 0
  n_chunks = b_per_w // CHUNK
  d_vecs = D // L

  mesh = plsc.VectorSubcoreMesh(core_axis_name="c", subcore_axis_name="s")

  @functools.partial(
      pl.kernel,
      out_type=jax.ShapeDtypeStruct((B, D), jnp.float32),
      mesh=mesh,
      scratch_types=[
          pltpu.VMEM((b_per_w,), jnp.int32),
          pltpu.VMEM((NBUF, CHUNK, D), jnp.float32),
          pltpu.SemaphoreType.DMA,
          pltpu.SemaphoreType.DMA,
          pltpu.SemaphoreType.DMA,
          pltpu.SemaphoreType.DMA,
      ],
      compiler_params=pltpu.CompilerParams(use_tc_tiling_on_sc=False),
  )
  def lookup(idx_hbm, table_hbm, out_hbm, idx_v, rows_v, g0, g1, s0, s1):
    gsem = (g0, g1)
    ssem = (s0, s1)
    wid = lax.axis_index("s") * NC + lax.axis_index("c")
    base = wid * b_per_w
    pltpu.sync_copy(idx_hbm.at[pl.ds(base, b_per_w)], idx_v)

    def gather(ch, b):
      return pltpu.make_async_copy(
          table_hbm.at[idx_v.at[pl.ds(ch * CHUNK, CHUNK)]],
          rows_v.at[b], gsem[b])

    def store(ch, b):
      return pltpu.make_async_copy(
          rows_v.at[b], out_hbm.at[pl.ds(base + ch * CHUNK, CHUNK)], ssem[b])

    for b in range(NBUF):
      gather(b, b).start()

    @pl.loop(0, n_chunks, step=NBUF)
    def _(g):
      for b in range(NBUF):
        ch = g + b
        gather(ch, b).wait()

        @plsc.parallel_loop(0, CHUNK, unroll=8)
        def _(i):
          for j in range(d_vecs):
            sl = pl.ds(j * L, L)
            rows_v[b, i, sl] = rows_v[b, i, sl] * SCALE

        store(ch, b).start()
        nxt = ch + NBUF

        @pl.when(nxt < n_chunks)
        def _():
          store(ch, b).wait()
          gather(nxt, b).start()

    for b in range(NBUF):
      store(n_chunks - NBUF + b, b).wait()

  return lookup


def kernel(x, table):
  S0, S1 = x.shape
  V, D = table.shape
  B = S0 * S1
  idx = x.reshape(B).astype(jnp.int32)
  out = _make_sc_lookup(B, D)(idx, table)
  return out.reshape(S0, S1, D)
